# baseline (speedup 1.0000x reference)
import functools
import math

import jax
import jax.numpy as jnp
from jax import lax
from jax.experimental import pallas as pl
from jax.experimental.pallas import tpu as pltpu
from jax.experimental.pallas import tpu_sc as plsc

BSZ, NODES, EDGES, D = 8, 512, 2048, 512
DH = D // 2
N_TOTAL = BSZ * NODES
E_TOTAL = BSZ * EDGES

_PROJ_BLK = 512


def _pack_halves(acc):
    lo = pltpu.bitcast(acc[:, :DH].astype(jnp.bfloat16), jnp.uint16)
    hi = pltpu.bitcast(acc[:, DH:].astype(jnp.bfloat16), jnp.uint16)
    word = (hi.astype(jnp.uint32) << 16) | lo.astype(jnp.uint32)
    return pltpu.bitcast(word, jnp.int32)


def _unpack_halves(word):
    wu = pltpu.bitcast(word, jnp.uint32)
    lo = pltpu.bitcast((wu & 0xFFFF).astype(jnp.uint16), jnp.bfloat16)
    hi = pltpu.bitcast((wu >> 16).astype(jnp.uint16), jnp.bfloat16)
    return lo.astype(jnp.float32), hi.astype(jnp.float32)


def _proj_body(x_ref, w_ref, out_ref):
    acc = jnp.dot(x_ref[...], w_ref[...], preferred_element_type=jnp.float32)
    out_ref[...] = _pack_halves(acc)


def _project_nodes(x, w1, wblk):
    nblk = N_TOTAL // _PROJ_BLK
    return pl.pallas_call(
        _proj_body,
        grid=(nblk,),
        in_specs=[
            pl.BlockSpec((_PROJ_BLK, D), lambda j: (j, 0)),
            pl.BlockSpec((D, D), lambda j: (wblk, 0)),
        ],
        out_specs=pl.BlockSpec((_PROJ_BLK, DH), lambda j: (j, 0)),
        out_shape=jax.ShapeDtypeStruct((N_TOTAL, DH), jnp.int32),
    )(x, w1)


_NC, _NS = 2, 16
_NW = _NC * _NS
_PER_TILE = E_TOTAL // _NW
_CHUNK = 128
_NCHUNK = _PER_TILE // _CHUNK


@functools.cache
def _sc_gather_kernel():
    mesh = plsc.VectorSubcoreMesh(core_axis_name="c", subcore_axis_name="s",
                                  num_cores=_NC, num_subcores=_NS)

    @functools.partial(
        pl.kernel,
        out_type=jax.ShapeDtypeStruct((E_TOTAL, DH), jnp.int32),
        mesh=mesh,
        scratch_types=[
            pltpu.VMEM((_PER_TILE,), jnp.int32),
            pltpu.VMEM((_CHUNK, DH), jnp.int32),
            pltpu.VMEM((_CHUNK, DH), jnp.int32),
            pltpu.SemaphoreType.DMA,
            pltpu.SemaphoreType.DMA,
        ],
    )
    def body(tbl_hbm, idx_hbm, out_hbm, idx_v, buf0, buf1, gsem, wsem):
        wid = lax.axis_index("s") * _NC + lax.axis_index("c")
        base = wid * _PER_TILE
        bufs = (buf0, buf1)
        pltpu.sync_copy(idx_hbm.at[pl.ds(base, _PER_TILE)], idx_v)
        gcp = {0: pltpu.async_copy(
            tbl_hbm.at[idx_v.at[pl.ds(0, _CHUNK)]], buf0, gsem)}
        wb = {}
        for c in range(_NCHUNK):
            if c + 1 < _NCHUNK:
                if c >= 1:
                    wb[c - 1].wait()
                gcp[c + 1] = pltpu.async_copy(
                    tbl_hbm.at[idx_v.at[pl.ds((c + 1) * _CHUNK, _CHUNK)]],
                    bufs[(c + 1) % 2], gsem)
            gcp[c].wait()
            wb[c] = pltpu.async_copy(
                bufs[c % 2], out_hbm.at[pl.ds(base + c * _CHUNK, _CHUNK)],
                wsem)
        wb[_NCHUNK - 2].wait()
        wb[_NCHUNK - 1].wait()

    return body


def _sc_gather(tbl, idx):
    return _sc_gather_kernel()(tbl, idx)


_MLP_BLK = 1024
_INV_SQRT2 = 1.0 / math.sqrt(2.0)


def _rel_body(r_ref, w1r_ref, out_ref):
    acc = jnp.dot(r_ref[...], w1r_ref[...], preferred_element_type=jnp.float32)
    out_ref[...] = _pack_halves(acc)


def _rel_proj(edge_rep, w1):
    eblk = E_TOTAL // _MLP_BLK
    return pl.pallas_call(
        _rel_body,
        grid=(eblk,),
        in_specs=[
            pl.BlockSpec((_MLP_BLK, D), lambda j: (j, 0)),
            pl.BlockSpec((D, D), lambda j: (1, 0)),
        ],
        out_specs=pl.BlockSpec((_MLP_BLK, DH), lambda j: (j, 0)),
        out_shape=jax.ShapeDtypeStruct((E_TOTAL, DH), jnp.int32),
    )(edge_rep, w1)


def _mlp_body(gh_ref, gt_ref, m_ref, b1_ref, gamma_ref, beta_ref,
              w2_ref, b2_ref, out_ref):
    hl, hh = _unpack_halves(gh_ref[...])
    tl, th = _unpack_halves(gt_ref[...])
    ml, mh = _unpack_halves(m_ref[...])
    h1 = jnp.concatenate([ml + hl + tl, mh + hh + th], axis=1) + b1_ref[...]
    mu = jnp.mean(h1, axis=-1, keepdims=True)
    var = jnp.mean((h1 - mu) ** 2, axis=-1, keepdims=True)
    h1n = (h1 - mu) * lax.rsqrt(var + 1e-5) * gamma_ref[...] + beta_ref[...]
    h1a = h1n * 0.5 * (1.0 + lax.erf(h1n * _INV_SQRT2))
    out_ref[...] = jnp.dot(h1a, w2_ref[...],
                           preferred_element_type=jnp.float32) + b2_ref[...]


def _mlp(gh, gt, m, b1, gamma, beta, w2, b2):
    eblk = E_TOTAL // _MLP_BLK
    vec = pl.BlockSpec((1, D), lambda j: (0, 0))
    return pl.pallas_call(
        _mlp_body,
        grid=(eblk,),
        in_specs=[
            pl.BlockSpec((_MLP_BLK, DH), lambda j: (j, 0)),
            pl.BlockSpec((_MLP_BLK, DH), lambda j: (j, 0)),
            pl.BlockSpec((_MLP_BLK, DH), lambda j: (j, 0)),
            vec, vec, vec,
            pl.BlockSpec((D, D), lambda j: (0, 0)),
            vec,
        ],
        out_specs=pl.BlockSpec((_MLP_BLK, D), lambda j: (j, 0)),
        out_shape=jax.ShapeDtypeStruct((E_TOTAL, D), jnp.float32),
    )(gh, gt, m, b1.reshape(1, D), gamma.reshape(1, D),
      beta.reshape(1, D), w2, b2.reshape(1, D))


def kernel(x, batch, edge_index, edge_rep, num_edges, ptr, W1, b1, gamma,
           beta, W2, b2):
    tbl_h = _project_nodes(x, W1, 0)
    gh = _sc_gather(tbl_h, edge_index[0])
    tbl_t = _project_nodes(x, W1, 2)
    gt = _sc_gather(tbl_t, edge_index[1])
    m = _rel_proj(edge_rep, W1)
    out = _mlp(gh, gt, m, b1, gamma, beta, W2, b2)
    mask = jnp.ones((BSZ, EDGES), dtype=jnp.float32)
    return out.reshape(BSZ, EDGES, D), mask


# speedup vs baseline: 1.0820x; 1.0820x over previous
import functools
import math

import jax
import jax.numpy as jnp
from jax import lax
from jax.experimental import pallas as pl
from jax.experimental.pallas import tpu as pltpu
from jax.experimental.pallas import tpu_sc as plsc

BSZ, NODES, EDGES, D = 8, 512, 2048, 512
DH = D // 2
N_TOTAL = BSZ * NODES
E_TOTAL = BSZ * EDGES
E_HALF = E_TOTAL // 2
G_HALF = 2 * E_HALF

_PROJ_BLK = 512


def _pack_halves(acc):
    lo = pltpu.bitcast(acc[:, :DH].astype(jnp.bfloat16), jnp.uint16)
    hi = pltpu.bitcast(acc[:, DH:].astype(jnp.bfloat16), jnp.uint16)
    word = (hi.astype(jnp.uint32) << 16) | lo.astype(jnp.uint32)
    return pltpu.bitcast(word, jnp.int32)


def _unpack_halves(word):
    wu = pltpu.bitcast(word, jnp.uint32)
    lo = pltpu.bitcast((wu & 0xFFFF).astype(jnp.uint16), jnp.bfloat16)
    hi = pltpu.bitcast((wu >> 16).astype(jnp.uint16), jnp.bfloat16)
    return lo.astype(jnp.float32), hi.astype(jnp.float32)


def _proj_body(x_ref, w_ref, out_ref):
    acc = jnp.dot(x_ref[...], w_ref[...], preferred_element_type=jnp.float32)
    out_ref[...] = _pack_halves(acc)


def _project_nodes(x, w1):
    nblk = N_TOTAL // _PROJ_BLK
    return pl.pallas_call(
        _proj_body,
        grid=(2 * nblk,),
        in_specs=[
            pl.BlockSpec((_PROJ_BLK, D), lambda j: (j % nblk, 0)),
            pl.BlockSpec((D, D), lambda j: (2 * (j // nblk), 0)),
        ],
        out_specs=pl.BlockSpec((_PROJ_BLK, DH), lambda j: (j, 0)),
        out_shape=jax.ShapeDtypeStruct((2 * N_TOTAL, DH), jnp.int32),
    )(x, w1)


_NC, _NS = 2, 16
_NW = _NC * _NS
_PER_TILE = G_HALF // _NW
_CHUNK = 128
_NCHUNK = _PER_TILE // _CHUNK


@functools.cache
def _sc_gather_kernel():
    mesh = plsc.VectorSubcoreMesh(core_axis_name="c", subcore_axis_name="s",
                                  num_cores=_NC, num_subcores=_NS)

    @functools.partial(
        pl.kernel,
        out_type=jax.ShapeDtypeStruct((G_HALF, DH), jnp.int32),
        mesh=mesh,
        scratch_types=[
            pltpu.VMEM((_PER_TILE,), jnp.int32),
            pltpu.VMEM((_CHUNK, DH), jnp.int32),
            pltpu.VMEM((_CHUNK, DH), jnp.int32),
            pltpu.SemaphoreType.DMA,
            pltpu.SemaphoreType.DMA,
        ],
    )
    def body(tbl_hbm, idx_hbm, out_hbm, idx_v, buf0, buf1, gsem, wsem):
        wid = lax.axis_index("s") * _NC + lax.axis_index("c")
        base = wid * _PER_TILE
        bufs = (buf0, buf1)
        pltpu.sync_copy(idx_hbm.at[pl.ds(base, _PER_TILE)], idx_v)
        gcp = {0: pltpu.async_copy(
            tbl_hbm.at[idx_v.at[pl.ds(0, _CHUNK)]], buf0, gsem)}
        wb = {}
        for c in range(_NCHUNK):
            if c + 1 < _NCHUNK:
                if c >= 1:
                    wb[c - 1].wait()
                gcp[c + 1] = pltpu.async_copy(
                    tbl_hbm.at[idx_v.at[pl.ds((c + 1) * _CHUNK, _CHUNK)]],
                    bufs[(c + 1) % 2], gsem)
            gcp[c].wait()
            wb[c] = pltpu.async_copy(
                bufs[c % 2], out_hbm.at[pl.ds(base + c * _CHUNK, _CHUNK)],
                wsem)
        wb[_NCHUNK - 2].wait()
        wb[_NCHUNK - 1].wait()

    return body


def _sc_gather(tbl, idx):
    return _sc_gather_kernel()(tbl, idx)


_MLP_BLK = 1024
_HBLK = E_HALF // _MLP_BLK
_INV_SQRT2 = 1.0 / math.sqrt(2.0)


def _mlp_math(gh, gt, r, w1r, b1, gamma, beta, w2, b2):
    m = jnp.dot(r, w1r, preferred_element_type=jnp.float32)
    hl, hh = _unpack_halves(gh)
    tl, th = _unpack_halves(gt)
    h1 = m + jnp.concatenate([hl + tl, hh + th], axis=1) + b1
    mu = jnp.mean(h1, axis=-1, keepdims=True)
    var = jnp.mean((h1 - mu) ** 2, axis=-1, keepdims=True)
    h1n = (h1 - mu) * lax.rsqrt(var + 1e-5) * gamma + beta
    h1a = h1n * 0.5 * (1.0 + lax.erf(h1n * _INV_SQRT2))
    return jnp.dot(h1a, w2, preferred_element_type=jnp.float32) + b2


def _mlp_body0(g_ref, r_ref, w1r_ref, b1_ref, gamma_ref, beta_ref,
               w2_ref, b2_ref, out_ref):
    out_ref[...] = _mlp_math(
        g_ref[0], g_ref[1], r_ref[...], w1r_ref[...], b1_ref[...],
        gamma_ref[...], beta_ref[...], w2_ref[...], b2_ref[...])


def _mlp_body1(prev_ref, g_ref, r_ref, w1r_ref, b1_ref, gamma_ref, beta_ref,
               w2_ref, b2_ref, out_ref):
    out_ref[...] = _mlp_math(
        g_ref[0], g_ref[1], r_ref[...], w1r_ref[...], b1_ref[...],
        gamma_ref[...], beta_ref[...], w2_ref[...], b2_ref[...])


def _mlp_half(g, edge_rep, w1, b1, gamma, beta, w2, b2, half, prev=None):
    vec = pl.BlockSpec((1, D), lambda j: (0, 0))
    gspec = pl.BlockSpec((2, _MLP_BLK, DH), lambda j: (0, j, 0))
    rspec = pl.BlockSpec((_MLP_BLK, D),
                         lambda j: (j + half * _HBLK, 0))
    out_spec = pl.BlockSpec((_MLP_BLK, D), lambda j: (j + half * _HBLK, 0))
    specs = [
        gspec, rspec,
        pl.BlockSpec((D, D), lambda j: (1, 0)),
        vec, vec, vec,
        pl.BlockSpec((D, D), lambda j: (0, 0)),
        vec,
    ]
    args = [g, edge_rep, w1, b1.reshape(1, D), gamma.reshape(1, D),
            beta.reshape(1, D), w2, b2.reshape(1, D)]
    if half == 0:
        body, alias = _mlp_body0, {}
    else:
        body, alias = _mlp_body1, {0: 0}
        specs = [pl.BlockSpec(memory_space=pl.ANY)] + specs
        args = [prev] + args
    return pl.pallas_call(
        body,
        grid=(_HBLK,),
        in_specs=specs,
        out_specs=out_spec,
        out_shape=jax.ShapeDtypeStruct((E_TOTAL, D), jnp.float32),
        input_output_aliases=alias,
    )(*args)


def kernel(x, batch, edge_index, edge_rep, num_edges, ptr, W1, b1, gamma,
           beta, W2, b2):
    tbl = _project_nodes(x, W1)
    src, dst = edge_index[0], edge_index[1] + N_TOTAL
    idx_a = jnp.concatenate([src[:E_HALF], dst[:E_HALF]])
    idx_b = jnp.concatenate([src[E_HALF:], dst[E_HALF:]])
    ga = _sc_gather(tbl, idx_a).reshape(2, E_HALF, DH)
    gb = _sc_gather(tbl, idx_b).reshape(2, E_HALF, DH)
    out = _mlp_half(ga, edge_rep, W1, b1, gamma, beta, W2, b2, 0)
    out = _mlp_half(gb, edge_rep, W1, b1, gamma, beta, W2, b2, 1, prev=out)
    mask = jnp.ones((BSZ, EDGES), dtype=jnp.float32)
    return out.reshape(BSZ, EDGES, D), mask
